# baseline (speedup 1.0000x reference)
import functools

import jax
import jax.numpy as jnp
from jax import lax
from jax.experimental import pallas as pl
from jax.experimental.pallas import tpu as pltpu
from jax.experimental.pallas import tpu_sc as plsc

_INFO = plsc.get_sparse_core_info()
_NC, _NS = _INFO.num_cores, _INFO.num_subcores
_NW = _NC * _NS


@functools.partial(jax.jit, static_argnames=("h", "b", "d"))
def _gather_t(x_t, table, *, h, b, d):
    blk = b // _NW
    tb = blk // 128
    te = d // 8
    mesh = plsc.VectorSubcoreMesh(core_axis_name="c", subcore_axis_name="s")

    @functools.partial(
        pl.kernel,
        out_type=jax.ShapeDtypeStruct((h, te, b // 128, 8, 128), jnp.float32),
        mesh=mesh,
        compiler_params=pltpu.CompilerParams(
            use_tc_tiling_on_sc=False, needs_layout_passes=False
        ),
        scratch_types=[
            pltpu.VMEM((h, blk), jnp.int32),
            [pltpu.VMEM((blk, d), jnp.float32) for _ in range(2)],
            pltpu.VMEM((te, tb, 8, 128), jnp.float32),
            [pltpu.SemaphoreType.DMA for _ in range(2)],
            pltpu.SemaphoreType.DMA,
        ],
    )
    def k(table_hbm, xt_hbm, out_hbm, idx_v, rows, trans_v, gsems, osem):
        wid = lax.axis_index("s") * _NC + lax.axis_index("c")
        base_b = wid * blk
        pltpu.sync_copy(xt_hbm.at[:, pl.ds(base_b, blk)], idx_v)
        iota = lax.iota(jnp.int32, 16)

        def start_g(c, r):
            pltpu.async_copy(table_hbm.at[idx_v.at[c]], rows[r], gsems[r])

        def wait_g(c, r):
            pltpu.make_async_copy(
                table_hbm.at[idx_v.at[c]], rows[r], gsems[r]
            ).wait()

        def start_o(c):
            pltpu.async_copy(
                trans_v, out_hbm.at[c, :, pl.ds(wid * tb, tb)], osem
            )

        def wait_o(c):
            pltpu.make_async_copy(
                trans_v, out_hbm.at[c, :, pl.ds(wid * tb, tb)], osem
            ).wait()

        def transpose(src):
            @pl.loop(0, tb * 8)
            def _blk(j):
                t = j >> 3
                bi0 = (j & 7) << 4
                bvec = iota + (t * 128 + bi0)

                @plsc.parallel_loop(0, d, unroll=8)
                def _e(e):
                    v = plsc.load_gather(src, [bvec, jnp.broadcast_to(e, (16,))])
                    trans_v[e >> 3, t, e & 7, pl.ds(bi0, 16)] = v

        start_g(0, 0)

        @pl.loop(0, h, step=2)
        def _pair(c):
            start_g(c + 1, 1)
            wait_g(c, 0)

            @pl.when(c > 0)
            def _():
                wait_o(c - 1)

            transpose(rows[0])
            start_o(c)

            @pl.when(c + 2 < h)
            def _():
                start_g(c + 2, 0)

            wait_g(c + 1, 1)
            wait_o(c)
            transpose(rows[1])
            start_o(c + 1)

        wait_o(h - 1)

    return k(table, x_t)


def kernel(x, table):
    b, h = x.shape
    _, d = table.shape
    x_t = jnp.transpose(x.astype(jnp.int32))
    p5 = _gather_t(x_t, table, h=h, b=b, d=d)
    return p5.transpose(2, 4, 0, 1, 3).reshape(b, h, d)


# speedup vs baseline: 1.1910x; 1.1910x over previous
import functools

import jax
import jax.numpy as jnp
from jax import lax
from jax.experimental import pallas as pl
from jax.experimental.pallas import tpu as pltpu
from jax.experimental.pallas import tpu_sc as plsc

_INFO = plsc.get_sparse_core_info()
_NC, _NS = _INFO.num_cores, _INFO.num_subcores
_NW = _NC * _NS


@functools.partial(jax.jit, static_argnames=("h", "b", "d"))
def _gather_t(x_t, table, *, h, b, d):
    blk = b // _NW
    tb = blk // 128
    te = d // 8
    mesh = plsc.VectorSubcoreMesh(core_axis_name="c", subcore_axis_name="s")

    @functools.partial(
        pl.kernel,
        out_type=jax.ShapeDtypeStruct((h, te, (b // 128) * 8, 128), jnp.float32),
        mesh=mesh,
        compiler_params=pltpu.CompilerParams(
            use_tc_tiling_on_sc=False, needs_layout_passes=False
        ),
        scratch_types=[
            pltpu.VMEM((h, blk), jnp.int32),
            [pltpu.VMEM((blk, d), jnp.float32) for _ in range(2)],
            pltpu.VMEM((te, tb * 8, 128), jnp.float32),
            [pltpu.SemaphoreType.DMA for _ in range(2)],
            pltpu.SemaphoreType.DMA,
        ],
    )
    def k(table_hbm, xt_hbm, out_hbm, idx_v, rows, trans_v, gsems, osem):
        wid = lax.axis_index("s") * _NC + lax.axis_index("c")
        base_b = wid * blk
        pltpu.sync_copy(xt_hbm.at[:, pl.ds(base_b, blk)], idx_v)
        iota = lax.iota(jnp.int32, 16)
        zv = jnp.zeros((16,), jnp.int32)
        mvecs = [(iota + kk) & 15 for kk in range(16)]
        dstat = [(m >> 3) * (tb * 8 * 128) + (m & 7) * 128 + iota for m in mvecs]

        def start_g(c, r):
            pltpu.async_copy(table_hbm.at[idx_v.at[c]], rows[r], gsems[r])

        def wait_g(c, r):
            pltpu.make_async_copy(
                table_hbm.at[idx_v.at[c]], rows[r], gsems[r]
            ).wait()

        def start_o(c):
            pltpu.async_copy(
                trans_v, out_hbm.at[c, :, pl.ds(wid * tb * 8, tb * 8)], osem
            )

        def wait_o(c):
            pltpu.make_async_copy(
                trans_v, out_hbm.at[c, :, pl.ds(wid * tb * 8, tb * 8)], osem
            ).wait()

        def transpose(src):
            @pl.loop(0, tb * 8)
            def _blk(j):
                t = j >> 3
                bi0 = (j & 7) << 4
                bvec = iota + (t * 128 + bi0)
                doff = t * 1024 + bi0

                for e0 in range(0, d, 16):
                    dbase = doff + (e0 >> 3) * (tb * 8 * 128)
                    for kk in range(16):
                        v = plsc.load_gather(src, [bvec, mvecs[kk] + e0])
                        plsc.store_scatter(
                            trans_v, [zv, zv, dstat[kk] + dbase], v
                        )

        start_g(0, 0)

        @pl.loop(0, h, step=2)
        def _pair(c):
            start_g(c + 1, 1)
            wait_g(c, 0)

            @pl.when(c > 0)
            def _():
                wait_o(c - 1)

            transpose(rows[0])
            start_o(c)

            @pl.when(c + 2 < h)
            def _():
                start_g(c + 2, 0)

            wait_g(c + 1, 1)
            wait_o(c)
            transpose(rows[1])
            start_o(c + 1)

        wait_o(h - 1)

    return k(table, x_t)


def kernel(x, table):
    b, h = x.shape
    _, d = table.shape
    x_t = jnp.transpose(x.astype(jnp.int32))
    p4 = _gather_t(x_t, table, h=h, b=b, d=d)
    p5 = p4.reshape(h, d // 8, b // 128, 8, 128)
    return p5.transpose(2, 4, 0, 1, 3).reshape(b, h, d)


# speedup vs baseline: 1.5160x; 1.2729x over previous
import functools

import jax
import jax.numpy as jnp
from jax import lax
from jax.experimental import pallas as pl
from jax.experimental.pallas import tpu as pltpu
from jax.experimental.pallas import tpu_sc as plsc

_INFO = plsc.get_sparse_core_info()
_NC, _NS = _INFO.num_cores, _INFO.num_subcores
_NW = _NC * _NS


@functools.partial(jax.jit, static_argnames=("h", "b", "d"))
def _gather_t(x_t, table, *, h, b, d):
    blk = b // _NW
    tb = blk // 128
    te = d // 8
    mesh = plsc.VectorSubcoreMesh(core_axis_name="c", subcore_axis_name="s")

    @functools.partial(
        pl.kernel,
        out_type=jax.ShapeDtypeStruct((h, te, (b // 128) * 8, 128), jnp.float32),
        mesh=mesh,
        compiler_params=pltpu.CompilerParams(
            use_tc_tiling_on_sc=False, needs_layout_passes=False
        ),
        scratch_types=[
            pltpu.VMEM((h, blk), jnp.int32),
            [pltpu.VMEM((blk, d), jnp.float32) for _ in range(2)],
            pltpu.VMEM((te, tb * 8, 128), jnp.float32),
            [pltpu.SemaphoreType.DMA for _ in range(2)],
            pltpu.SemaphoreType.DMA,
        ],
    )
    def k(table_hbm, xt_hbm, out_hbm, idx_v, rows, trans_v, gsems, osem):
        wid = lax.axis_index("s") * _NC + lax.axis_index("c")
        base_b = wid * blk
        pltpu.sync_copy(xt_hbm.at[:, pl.ds(base_b, blk)], idx_v)
        iota = lax.iota(jnp.int32, 16)
        zv = jnp.zeros((16,), jnp.int32)
        mvecs = [(iota + kk) & 15 for kk in range(16)]
        dstat = [(m >> 3) * (tb * 8 * 128) + (m & 7) * 128 + iota for m in mvecs]

        def start_g(c, r):
            pltpu.async_copy(table_hbm.at[idx_v.at[c]], rows[r], gsems[r])

        def wait_g(c, r):
            pltpu.make_async_copy(
                table_hbm.at[idx_v.at[c]], rows[r], gsems[r]
            ).wait()

        def start_o(c):
            pltpu.async_copy(
                trans_v, out_hbm.at[c, :, pl.ds(wid * tb * 8, tb * 8)], osem
            )

        def wait_o(c):
            pltpu.make_async_copy(
                trans_v, out_hbm.at[c, :, pl.ds(wid * tb * 8, tb * 8)], osem
            ).wait()

        def transpose(src):
            @pl.loop(0, tb * 8)
            def _blk(j):
                t = j >> 3
                bi0 = (j & 7) << 4
                bvec = iota + (t * 128 + bi0)
                doff = t * 1024 + bi0

                @plsc.parallel_loop(0, d // 16, unroll=2)
                def _e0(eo4):
                    e0 = eo4 * 16
                    dbase = doff + eo4 * 2 * (tb * 8 * 128)
                    for kk in range(16):
                        v = plsc.load_gather(src, [bvec, mvecs[kk] + e0])
                        plsc.store_scatter(
                            trans_v, [zv, zv, dstat[kk] + dbase], v
                        )

        start_g(0, 0)

        @pl.loop(0, h, step=2)
        def _pair(c):
            start_g(c + 1, 1)
            wait_g(c, 0)

            @pl.when(c > 0)
            def _():
                wait_o(c - 1)

            transpose(rows[0])
            start_o(c)

            @pl.when(c + 2 < h)
            def _():
                start_g(c + 2, 0)

            wait_g(c + 1, 1)
            wait_o(c)
            transpose(rows[1])
            start_o(c + 1)

        wait_o(h - 1)

    return k(table, x_t)


def kernel(x, table):
    b, h = x.shape
    _, d = table.shape
    x_t = jnp.transpose(x.astype(jnp.int32))
    p4 = _gather_t(x_t, table, h=h, b=b, d=d)
    p5 = p4.reshape(h, d // 8, b // 128, 8, 128)
    return p5.transpose(2, 4, 0, 1, 3).reshape(b, h, d)


# speedup vs baseline: 1.6303x; 1.0754x over previous
import functools

import jax
import jax.numpy as jnp
from jax import lax
from jax.experimental import pallas as pl
from jax.experimental.pallas import tpu as pltpu
from jax.experimental.pallas import tpu_sc as plsc

_INFO = plsc.get_sparse_core_info()
_NC, _NS = _INFO.num_cores, _INFO.num_subcores
_NW = _NC * _NS


@functools.partial(jax.jit, static_argnames=("h", "b", "d"))
def _gather_t(x_t, table, *, h, b, d):
    blk = b // _NW
    ch = blk // 2
    tc = ch // 128
    te = d // 8
    n = 2 * h
    mesh = plsc.VectorSubcoreMesh(core_axis_name="c", subcore_axis_name="s")

    @functools.partial(
        pl.kernel,
        out_type=jax.ShapeDtypeStruct((h, te, (b // 128) * 8, 128), jnp.float32),
        mesh=mesh,
        compiler_params=pltpu.CompilerParams(
            use_tc_tiling_on_sc=False, needs_layout_passes=False
        ),
        scratch_types=[
            pltpu.VMEM((h, blk), jnp.int32),
            [pltpu.VMEM((ch, d), jnp.float32) for _ in range(2)],
            [pltpu.VMEM((te, tc * 8, 128), jnp.float32) for _ in range(2)],
            [pltpu.SemaphoreType.DMA for _ in range(2)],
            [pltpu.SemaphoreType.DMA for _ in range(2)],
        ],
    )
    def k(table_hbm, xt_hbm, out_hbm, idx_v, rows, trans, gsems, osems):
        wid = lax.axis_index("s") * _NC + lax.axis_index("c")
        pltpu.sync_copy(xt_hbm.at[:, pl.ds(wid * blk, blk)], idx_v)
        iota = lax.iota(jnp.int32, 16)
        zv = jnp.zeros((16,), jnp.int32)
        mvecs = [(iota + kk) & 15 for kk in range(16)]
        dstat = [(m >> 3) * (tc * 8 * 128) + (m & 7) * 128 + iota for m in mvecs]

        def _idx(c):
            return idx_v.at[c >> 1, pl.ds((c & 1) * ch, ch)]

        def _out(c):
            row0 = wid * (2 * tc) * 8 + (c & 1) * tc * 8
            return out_hbm.at[c >> 1, :, pl.ds(row0, tc * 8)]

        def start_g(c, r):
            pltpu.async_copy(table_hbm.at[_idx(c)], rows[r], gsems[r])

        def wait_g(c, r):
            pltpu.make_async_copy(table_hbm.at[_idx(c)], rows[r], gsems[r]).wait()

        def start_o(c, r):
            pltpu.async_copy(trans[r], _out(c), osems[r])

        def wait_o(c, r):
            pltpu.make_async_copy(trans[r], _out(c), osems[r]).wait()

        def transpose(src, dst):
            @pl.loop(0, tc * 8)
            def _blk(j):
                t = j >> 3
                bi0 = (j & 7) << 4
                bvec = iota + (t * 128 + bi0)
                doff = t * 1024 + bi0

                @plsc.parallel_loop(0, d // 16, unroll=2)
                def _e0(eo4):
                    e0 = eo4 * 16
                    dbase = doff + eo4 * 2 * (tc * 8 * 128)
                    for kk in range(16):
                        v = plsc.load_gather(src, [bvec, mvecs[kk] + e0])
                        plsc.store_scatter(dst, [zv, zv, dstat[kk] + dbase], v)

        start_g(0, 0)
        start_g(1, 1)

        @pl.loop(0, n, step=2)
        def _pair(c):
            wait_g(c, 0)

            @pl.when(c > 1)
            def _():
                wait_o(c - 2, 0)

            transpose(rows[0], trans[0])
            start_o(c, 0)

            @pl.when(c + 2 < n)
            def _():
                start_g(c + 2, 0)

            wait_g(c + 1, 1)

            @pl.when(c > 1)
            def _():
                wait_o(c - 1, 1)

            transpose(rows[1], trans[1])
            start_o(c + 1, 1)

            @pl.when(c + 3 < n)
            def _():
                start_g(c + 3, 1)

        wait_o(n - 2, 0)
        wait_o(n - 1, 1)

    return k(table, x_t)


def kernel(x, table):
    b, h = x.shape
    _, d = table.shape
    x_t = jnp.transpose(x.astype(jnp.int32))
    p4 = _gather_t(x_t, table, h=h, b=b, d=d)
    p5 = p4.reshape(h, d // 8, b // 128, 8, 128)
    return p5.transpose(2, 4, 0, 1, 3).reshape(b, h, d)


# speedup vs baseline: 2.9891x; 1.8334x over previous
import functools

import jax
import jax.numpy as jnp
from jax import lax
from jax.experimental import pallas as pl
from jax.experimental.pallas import tpu as pltpu
from jax.experimental.pallas import tpu_sc as plsc

_INFO = plsc.get_sparse_core_info()
_NC, _NS = _INFO.num_cores, _INFO.num_subcores
_NW = _NC * _NS


@functools.partial(jax.jit, static_argnames=("h", "b", "d"))
def _gather_t(x_t, table, *, h, b, d):
    blk = b // _NW
    ch = blk // 2
    tc = ch // 128
    te = d // 8
    n = 2 * h
    mesh = plsc.VectorSubcoreMesh(core_axis_name="c", subcore_axis_name="s")

    @functools.partial(
        pl.kernel,
        out_type=jax.ShapeDtypeStruct((h, te, (b // 128) * 8, 128), jnp.float32),
        mesh=mesh,
        compiler_params=pltpu.CompilerParams(
            use_tc_tiling_on_sc=False, needs_layout_passes=False
        ),
        scratch_types=[
            pltpu.VMEM((h, blk), jnp.int32),
            [pltpu.VMEM((ch, d), jnp.float32) for _ in range(2)],
            [pltpu.VMEM((te, tc * 8, 128), jnp.float32) for _ in range(2)],
            [pltpu.SemaphoreType.DMA for _ in range(2)],
            [pltpu.SemaphoreType.DMA for _ in range(2)],
        ],
    )
    def k(table_hbm, xt_hbm, out_hbm, idx_v, rows, trans, gsems, osems):
        wid = lax.axis_index("s") * _NC + lax.axis_index("c")
        pltpu.sync_copy(xt_hbm.at[:, pl.ds(wid * blk, blk)], idx_v)
        iota = lax.iota(jnp.int32, 16)
        zv = jnp.zeros((16,), jnp.int32)
        mvecs = [(iota + kk) & 15 for kk in range(16)]
        dstat = [(m >> 3) * (tc * 8 * 128) + (m & 7) * 128 + iota for m in mvecs]

        def _idx(c):
            return idx_v.at[c >> 1, pl.ds((c & 1) * ch, ch)]

        def _out(c):
            row0 = wid * (2 * tc) * 8 + (c & 1) * tc * 8
            return out_hbm.at[c >> 1, :, pl.ds(row0, tc * 8)]

        def start_g(c, r):
            pltpu.async_copy(table_hbm.at[_idx(c)], rows[r], gsems[r])

        def wait_g(c, r):
            pltpu.make_async_copy(table_hbm.at[_idx(c)], rows[r], gsems[r]).wait()

        def start_o(c, r):
            pltpu.async_copy(trans[r], _out(c), osems[r])

        def wait_o(c, r):
            pltpu.make_async_copy(trans[r], _out(c), osems[r]).wait()

        def transpose(src, dst):
            @pl.loop(0, tc * 8)
            def _blk(j):
                t = j >> 3
                bi0 = (j & 7) << 4
                bvec = iota + (t * 128 + bi0)
                doff = t * 1024 + bi0

                @plsc.parallel_loop(0, d // 16, unroll=2)
                def _e0(eo4):
                    e0 = eo4 * 16
                    dbase = doff + eo4 * 2 * (tc * 8 * 128)
                    for kk in range(16):
                        v = plsc.load_gather(src, [bvec, mvecs[kk] + e0])
                        plsc.store_scatter(dst, [zv, zv, dstat[kk] + dbase], v)

        start_g(0, 0)
        start_g(1, 1)

        @pl.loop(0, n, step=2)
        def _pair(c):
            wait_g(c, 0)

            @pl.when(c > 1)
            def _():
                wait_o(c - 2, 0)

            transpose(rows[0], trans[0])
            start_o(c, 0)

            @pl.when(c + 2 < n)
            def _():
                start_g(c + 2, 0)

            wait_g(c + 1, 1)

            @pl.when(c > 1)
            def _():
                wait_o(c - 1, 1)

            transpose(rows[1], trans[1])
            start_o(c + 1, 1)

            @pl.when(c + 3 < n)
            def _():
                start_g(c + 3, 1)

        wait_o(n - 2, 0)
        wait_o(n - 1, 1)

    return k(table, x_t)


@functools.partial(jax.jit, static_argnames=("v", "d"))
def _detile(tview, *, v, d):
    nt = (v + 127) // 128
    per_w = -(-nt // _NW)
    per_w += per_w & 1
    mesh = plsc.VectorSubcoreMesh(core_axis_name="c", subcore_axis_name="s")

    @functools.partial(
        pl.kernel,
        out_type=jax.ShapeDtypeStruct((v // 2, 2 * d), jnp.float32),
        mesh=mesh,
        compiler_params=pltpu.CompilerParams(
            use_tc_tiling_on_sc=True,
            needs_layout_passes=False,
            disable_bounds_checks=True
        ),
        scratch_types=[
            [pltpu.VMEM((d, 128), jnp.float32) for _ in range(2)],
            [pltpu.VMEM((d, 128), jnp.float32) for _ in range(2)],
            [pltpu.SemaphoreType.DMA for _ in range(2)],
            [pltpu.SemaphoreType.DMA for _ in range(2)],
        ],
    )
    def k(tv_hbm, out_hbm, bufs, trans, isems, osems):
        wid = lax.axis_index("s") * _NC + lax.axis_index("c")
        iota = lax.iota(jnp.int32, 16)
        zv = jnp.zeros((16,), jnp.int32)
        mvecs = [(iota + kk) & 15 for kk in range(16)]
        sstat = [(m >> 3) * (16 * d) + (m & 7) * 128 + iota for m in mvecs]
        dstat = [iota * d + m for m in mvecs]

        def tj_of(i):
            return i * _NW + wid

        def start_i(i, r):
            pltpu.async_copy(
                tv_hbm.at[:, pl.ds(tj_of(i) * 128, 128)], bufs[r], isems[r]
            )

        def wait_i(i, r):
            pltpu.make_async_copy(
                tv_hbm.at[:, pl.ds(tj_of(i) * 128, 128)], bufs[r], isems[r]
            ).wait()

        def start_o(i, r):
            tj = tj_of(i)

            @pl.when(tj < nt - 1)
            def _():
                pltpu.async_copy(trans[r], out_hbm.at[pl.ds(tj * d, d)], osems[r])

            @pl.when(tj == nt - 1)
            def _():
                pltpu.async_copy(
                    trans[r].at[pl.ds(0, (v - (nt - 1) * 128) // 2)],
                    out_hbm.at[pl.ds(tj * d, (v - (nt - 1) * 128) // 2)],
                    osems[r],
                )

        def wait_o(i, r):
            tj = tj_of(i)

            @pl.when(tj < nt - 1)
            def _():
                pltpu.make_async_copy(
                    trans[r], out_hbm.at[pl.ds(tj * d, d)], osems[r]
                ).wait()

            @pl.when(tj == nt - 1)
            def _():
                pltpu.make_async_copy(
                    trans[r].at[pl.ds(0, (v - (nt - 1) * 128) // 2)],
                    out_hbm.at[pl.ds(tj * d, (v - (nt - 1) * 128) // 2)],
                    osems[r],
                ).wait()

        def transpose(src, dst):
            @plsc.parallel_loop(0, 32, unroll=2)
            def _b(j2):
                v0 = (j2 >> 2) << 4
                a = j2 & 3
                sbase = a * (2 * 16 * d) + v0
                dbase = v0 * d + a * 16
                for kk in range(16):
                    val = plsc.load_gather(src, [zv, sstat[kk] + sbase])
                    plsc.store_scatter(dst, [zv, dstat[kk] + dbase], val)

        def live(i):
            return tj_of(i) < nt

        @pl.when(live(0))
        def _():
            start_i(0, 0)

        @pl.when(live(1))
        def _():
            start_i(1, 1)

        @pl.loop(0, per_w, step=2)
        def _pair(i):
            for r in range(2):
                ii = i + r

                @pl.when(live(ii))
                def _():
                    wait_i(ii, r)

                    @pl.when((ii > 1) & live(ii - 2))
                    def _():
                        wait_o(ii - 2, r)

                    transpose(bufs[r], trans[r])
                    start_o(ii, r)

                    @pl.when(live(ii + 2))
                    def _():
                        start_i(ii + 2, r)

        n_live = lax.div(nt - wid + (_NW - 1), _NW)
        i_last = n_live - 1
        for back in range(2):
            il = i_last - back
            for r in range(2):

                @pl.when((il >= 0) & (lax.rem(il, 2) == r))
                def _():
                    wait_o(il, r)

    return k(tview)


def kernel(x, table):
    b, h = x.shape
    _, d = table.shape
    x_t = jnp.transpose(x.astype(jnp.int32))
    v = table.shape[0]
    lin = _detile(jnp.transpose(table), v=v, d=d)
    table_lin = lin.reshape(v, d)
    p4 = _gather_t(x_t, table_lin, h=h, b=b, d=d)
    p5 = p4.reshape(h, d // 8, b // 128, 8, 128)
    return p5.transpose(2, 4, 0, 1, 3).reshape(b, h, d)


# speedup vs baseline: 3.2661x; 1.0927x over previous
import functools

import jax
import jax.numpy as jnp
from jax import lax
from jax.experimental import pallas as pl
from jax.experimental.pallas import tpu as pltpu
from jax.experimental.pallas import tpu_sc as plsc

_INFO = plsc.get_sparse_core_info()
_NC, _NS = _INFO.num_cores, _INFO.num_subcores
_NW = _NC * _NS


@functools.partial(jax.jit, static_argnames=("h", "b", "d"))
def _gather_t(x_t, table, *, h, b, d):
    blk = b // _NW
    ch = blk // 2
    tc = ch // 128
    te = d // 8
    n = 2 * h
    mesh = plsc.VectorSubcoreMesh(core_axis_name="c", subcore_axis_name="s")

    @functools.partial(
        pl.kernel,
        out_type=jax.ShapeDtypeStruct((h, te, (b // 128) * 8, 128), jnp.float32),
        mesh=mesh,
        compiler_params=pltpu.CompilerParams(
            use_tc_tiling_on_sc=False, needs_layout_passes=False
        ),
        scratch_types=[
            pltpu.VMEM((h, blk), jnp.int32),
            [pltpu.VMEM((ch, d), jnp.float32) for _ in range(2)],
            [pltpu.VMEM((te, tc * 8, 128), jnp.float32) for _ in range(2)],
            [pltpu.SemaphoreType.DMA for _ in range(2)],
            [pltpu.SemaphoreType.DMA for _ in range(2)],
        ],
    )
    def k(table_hbm, xt_hbm, out_hbm, idx_v, rows, trans, gsems, osems):
        wid = lax.axis_index("s") * _NC + lax.axis_index("c")
        pltpu.sync_copy(xt_hbm.at[:, pl.ds(wid * blk, blk)], idx_v)
        iota = lax.iota(jnp.int32, 16)
        zv = jnp.zeros((16,), jnp.int32)
        mvecs = [(iota + kk) & 15 for kk in range(16)]
        dstat = [(m >> 3) * (tc * 8 * 128) + (m & 7) * 128 + iota for m in mvecs]

        def _idx(c):
            return idx_v.at[c >> 1, pl.ds((c & 1) * ch, ch)]

        def _out(c):
            row0 = wid * (2 * tc) * 8 + (c & 1) * tc * 8
            return out_hbm.at[c >> 1, :, pl.ds(row0, tc * 8)]

        def start_g(c, r):
            pltpu.async_copy(table_hbm.at[_idx(c)], rows[r], gsems[r])

        def wait_g(c, r):
            pltpu.make_async_copy(table_hbm.at[_idx(c)], rows[r], gsems[r]).wait()

        def start_o(c, r):
            pltpu.async_copy(trans[r], _out(c), osems[r])

        def wait_o(c, r):
            pltpu.make_async_copy(trans[r], _out(c), osems[r]).wait()

        def transpose(src, dst):
            @pl.loop(0, tc * 8)
            def _blk(j):
                t = j >> 3
                bi0 = (j & 7) << 4
                bvec = iota + (t * 128 + bi0)
                doff = t * 1024 + bi0

                @plsc.parallel_loop(0, d // 16, unroll=4)
                def _e0(eo4):
                    e0 = eo4 * 16
                    dbase = doff + eo4 * 2 * (tc * 8 * 128)
                    for kk in range(16):
                        v = plsc.load_gather(src, [bvec, mvecs[kk] + e0])
                        plsc.store_scatter(dst, [zv, zv, dstat[kk] + dbase], v)

        start_g(0, 0)
        start_g(1, 1)

        @pl.loop(0, n, step=2)
        def _pair(c):
            wait_g(c, 0)

            @pl.when(c > 1)
            def _():
                wait_o(c - 2, 0)

            transpose(rows[0], trans[0])
            start_o(c, 0)

            @pl.when(c + 2 < n)
            def _():
                start_g(c + 2, 0)

            wait_g(c + 1, 1)

            @pl.when(c > 1)
            def _():
                wait_o(c - 1, 1)

            transpose(rows[1], trans[1])
            start_o(c + 1, 1)

            @pl.when(c + 3 < n)
            def _():
                start_g(c + 3, 1)

        wait_o(n - 2, 0)
        wait_o(n - 1, 1)

    return k(table, x_t)


@functools.partial(jax.jit, static_argnames=("v", "d"))
def _detile(tview, *, v, d):
    nt = (v + 127) // 128
    per_w = -(-nt // _NW)
    per_w += per_w & 1
    mesh = plsc.VectorSubcoreMesh(core_axis_name="c", subcore_axis_name="s")

    @functools.partial(
        pl.kernel,
        out_type=jax.ShapeDtypeStruct((v // 2, 2 * d), jnp.float32),
        mesh=mesh,
        compiler_params=pltpu.CompilerParams(
            use_tc_tiling_on_sc=True,
            needs_layout_passes=False,
            disable_bounds_checks=True
        ),
        scratch_types=[
            [pltpu.VMEM((d, 128), jnp.float32) for _ in range(2)],
            [pltpu.VMEM((d, 128), jnp.float32) for _ in range(2)],
            [pltpu.SemaphoreType.DMA for _ in range(2)],
            [pltpu.SemaphoreType.DMA for _ in range(2)],
        ],
    )
    def k(tv_hbm, out_hbm, bufs, trans, isems, osems):
        wid = lax.axis_index("s") * _NC + lax.axis_index("c")
        iota = lax.iota(jnp.int32, 16)
        zv = jnp.zeros((16,), jnp.int32)
        mvecs = [(iota + kk) & 15 for kk in range(16)]
        sstat = [(m >> 3) * (16 * d) + (m & 7) * 128 + iota for m in mvecs]
        dstat = [iota * d + m for m in mvecs]

        def tj_of(i):
            return i * _NW + wid

        def start_i(i, r):
            pltpu.async_copy(
                tv_hbm.at[:, pl.ds(tj_of(i) * 128, 128)], bufs[r], isems[r]
            )

        def wait_i(i, r):
            pltpu.make_async_copy(
                tv_hbm.at[:, pl.ds(tj_of(i) * 128, 128)], bufs[r], isems[r]
            ).wait()

        def start_o(i, r):
            tj = tj_of(i)

            @pl.when(tj < nt - 1)
            def _():
                pltpu.async_copy(trans[r], out_hbm.at[pl.ds(tj * d, d)], osems[r])

            @pl.when(tj == nt - 1)
            def _():
                pltpu.async_copy(
                    trans[r].at[pl.ds(0, (v - (nt - 1) * 128) // 2)],
                    out_hbm.at[pl.ds(tj * d, (v - (nt - 1) * 128) // 2)],
                    osems[r],
                )

        def wait_o(i, r):
            tj = tj_of(i)

            @pl.when(tj < nt - 1)
            def _():
                pltpu.make_async_copy(
                    trans[r], out_hbm.at[pl.ds(tj * d, d)], osems[r]
                ).wait()

            @pl.when(tj == nt - 1)
            def _():
                pltpu.make_async_copy(
                    trans[r].at[pl.ds(0, (v - (nt - 1) * 128) // 2)],
                    out_hbm.at[pl.ds(tj * d, (v - (nt - 1) * 128) // 2)],
                    osems[r],
                ).wait()

        def transpose(src, dst):
            @plsc.parallel_loop(0, 32, unroll=4)
            def _b(j2):
                v0 = (j2 >> 2) << 4
                a = j2 & 3
                sbase = a * (2 * 16 * d) + v0
                dbase = v0 * d + a * 16
                for kk in range(16):
                    val = plsc.load_gather(src, [zv, sstat[kk] + sbase])
                    plsc.store_scatter(dst, [zv, dstat[kk] + dbase], val)

        def live(i):
            return tj_of(i) < nt

        @pl.when(live(0))
        def _():
            start_i(0, 0)

        @pl.when(live(1))
        def _():
            start_i(1, 1)

        @pl.loop(0, per_w, step=2)
        def _pair(i):
            for r in range(2):
                ii = i + r

                @pl.when(live(ii))
                def _():
                    wait_i(ii, r)

                    @pl.when((ii > 1) & live(ii - 2))
                    def _():
                        wait_o(ii - 2, r)

                    transpose(bufs[r], trans[r])
                    start_o(ii, r)

                    @pl.when(live(ii + 2))
                    def _():
                        start_i(ii + 2, r)

        n_live = lax.div(nt - wid + (_NW - 1), _NW)
        i_last = n_live - 1
        for back in range(2):
            il = i_last - back
            for r in range(2):

                @pl.when((il >= 0) & (lax.rem(il, 2) == r))
                def _():
                    wait_o(il, r)

    return k(tview)


def kernel(x, table):
    b, h = x.shape
    _, d = table.shape
    x_t = jnp.transpose(x.astype(jnp.int32))
    v = table.shape[0]
    lin = _detile(jnp.transpose(table), v=v, d=d)
    table_lin = lin.reshape(v, d)
    p4 = _gather_t(x_t, table_lin, h=h, b=b, d=d)
    p5 = p4.reshape(h, d // 8, b // 128, 8, 128)
    return p5.transpose(2, 4, 0, 1, 3).reshape(b, h, d)
